# Initial kernel scaffold; baseline (speedup 1.0000x reference)
import jax
import jax.numpy as jnp
from jax.experimental import pallas as pl


def kernel(batch_size, total_length, position_ids, week_ids, day_ids, hour_ids, minute_ids, device, position_table, week_table, day_table, hour_table, minute_table):
    raise NotImplementedError("write your pallas kernel here")


# speedup vs baseline: 4.4597x; 4.4597x over previous
import functools

import jax
import jax.numpy as jnp
from jax import lax
from jax.experimental import pallas as pl
from jax.experimental.pallas import tpu as pltpu
from jax.experimental.pallas import tpu_sc as plsc

_NC = 2
_NS = 16
_LANES = 16


def _hm_body(h_ref, m_ref, o_ref):
    o_ref[...] = h_ref[...][:, None, :] + m_ref[...][None, :, :]


def _build_hm(hour_table, minute_table):
    nh, d = hour_table.shape
    nm = minute_table.shape[0]
    hm = pl.pallas_call(
        _hm_body,
        out_shape=jax.ShapeDtypeStruct((nh, nm, d), jnp.float32),
    )(hour_table, minute_table)
    return hm.reshape(nh * nm, d)


def _make_sc_kernel(B, T, S, D, NM):
    NW = _NC * _NS
    SG = 4
    PG = NW // SG
    SPW = S // SG
    BPW = B // PG
    NVEC = SPW // _LANES
    DVEC = D // _LANES

    mesh = plsc.VectorSubcoreMesh(
        core_axis_name="c", subcore_axis_name="s",
        num_cores=_NC, num_subcores=_NS)

    @functools.partial(
        pl.kernel,
        out_type=jax.ShapeDtypeStruct((B, T, S, D), jnp.float32),
        mesh=mesh,
        scratch_types=[
            pltpu.VMEM((BPW, T, SPW), jnp.int32),
            pltpu.VMEM((BPW, T, SPW), jnp.int32),
            pltpu.VMEM((BPW, T, SPW), jnp.int32),
            pltpu.VMEM((SPW, D), jnp.float32),
            pltpu.VMEM((SPW, D), jnp.float32),
            pltpu.SemaphoreType.DMA,
        ],
    )
    def sc_kernel(pos_hbm, h_hbm, m_hbm, hm_hbm, out_hbm,
                  h_v, m_v, c_v, pos_v, rows_v, sem):
        cid = lax.axis_index("c")
        sid = lax.axis_index("s")
        wid = sid * _NC + cid
        sg = wid % SG
        pg = wid // SG
        s0 = sg * SPW
        b0 = pg * BPW

        pltpu.sync_copy(pos_hbm.at[pl.ds(s0, SPW), :], pos_v)
        for bl in range(BPW):
            pltpu.sync_copy(h_hbm.at[b0 + bl, :, pl.ds(s0, SPW)], h_v.at[bl])
            pltpu.sync_copy(m_hbm.at[b0 + bl, :, pl.ds(s0, SPW)], m_v.at[bl])

        def cid_body(i, carry):
            bl = i // T
            t = i % T
            for j in range(NVEC):
                sl = pl.ds(j * _LANES, _LANES)
                c_v[bl, t, sl] = h_v[bl, t, sl] * NM + m_v[bl, t, sl]
            return carry
        lax.fori_loop(0, BPW * T, cid_body, 0)

        def pair_body(i, carry):
            bl = i // T
            t = i % T
            pltpu.async_copy(hm_hbm.at[c_v.at[bl, t]], rows_v, sem).wait()

            def add_body(r, c2):
                for j in range(DVEC):
                    sl = pl.ds(j * _LANES, _LANES)
                    rows_v[r, sl] = rows_v[r, sl] + pos_v[r, sl]
                return c2
            lax.fori_loop(0, SPW, add_body, 0)

            pltpu.sync_copy(rows_v, out_hbm.at[b0 + bl, t, pl.ds(s0, SPW), :])
            return carry
        lax.fori_loop(0, BPW * T, pair_body, 0)

    return sc_kernel


def kernel(batch_size, total_length, position_ids, week_ids, day_ids,
           hour_ids, minute_ids, device, position_table, week_table,
           day_table, hour_table, minute_table):
    S, D = position_table.shape
    B = hour_ids.shape[0]
    T = hour_ids.shape[1] // S
    NM = minute_table.shape[0]

    hm = _build_hm(hour_table, minute_table)
    h3 = hour_ids.reshape(B, T, S)
    m3 = minute_ids.reshape(B, T, S)

    sc = _make_sc_kernel(B, T, S, D, NM)
    return sc(position_table, h3, m3, hm)


# speedup vs baseline: 4.5450x; 1.0191x over previous
import functools

import jax
import jax.numpy as jnp
from jax import lax
from jax.experimental import pallas as pl
from jax.experimental.pallas import tpu as pltpu
from jax.experimental.pallas import tpu_sc as plsc

_NC = 2
_NS = 16
_LANES = 16


def _hm_body(h_ref, m_ref, o_ref):
    o_ref[...] = h_ref[...][:, None, :] + m_ref[...][None, :, :]


def _build_hm(hour_table, minute_table):
    nh, d = hour_table.shape
    nm = minute_table.shape[0]
    hm = pl.pallas_call(
        _hm_body,
        out_shape=jax.ShapeDtypeStruct((nh, nm, d), jnp.float32),
    )(hour_table, minute_table)
    return hm.reshape(nh * nm, d)


def _make_sc_kernel(B, T, S, D, NM):
    NW = _NC * _NS
    SG = 4
    PG = NW // SG
    SPW = S // SG
    BPW = B // PG
    NP = BPW * T
    NVEC = SPW // _LANES
    DVEC = D // _LANES
    NBUF = 3

    mesh = plsc.VectorSubcoreMesh(
        core_axis_name="c", subcore_axis_name="s",
        num_cores=_NC, num_subcores=_NS)

    @functools.partial(
        pl.kernel,
        out_type=jax.ShapeDtypeStruct((B, T, S, D), jnp.float32),
        mesh=mesh,
        scratch_types=[
            pltpu.VMEM((BPW, T, SPW), jnp.int32),
            pltpu.VMEM((BPW, T, SPW), jnp.int32),
            pltpu.VMEM((BPW, T, SPW), jnp.int32),
            pltpu.VMEM((SPW, D), jnp.float32),
            [pltpu.VMEM((SPW, D), jnp.float32)] * NBUF,
            [pltpu.SemaphoreType.DMA] * NBUF,
            [pltpu.SemaphoreType.DMA] * NBUF,
        ],
    )
    def sc_kernel(pos_hbm, h_hbm, m_hbm, hm_hbm, out_hbm,
                  h_v, m_v, c_v, pos_v, rows, gsem, osem):
        cid = lax.axis_index("c")
        sid = lax.axis_index("s")
        wid = sid * _NC + cid
        sg = wid % SG
        pg = wid // SG
        s0 = sg * SPW
        b0 = pg * BPW

        pltpu.sync_copy(pos_hbm.at[pl.ds(s0, SPW), :], pos_v)
        for bl in range(BPW):
            pltpu.sync_copy(h_hbm.at[b0 + bl, :, pl.ds(s0, SPW)], h_v.at[bl])
            pltpu.sync_copy(m_hbm.at[b0 + bl, :, pl.ds(s0, SPW)], m_v.at[bl])

        @plsc.parallel_loop(0, BPW * T)
        def cid_body(i):
            bl = i // T
            t = i % T
            for j in range(NVEC):
                sl = pl.ds(j * _LANES, _LANES)
                c_v[bl, t, sl] = h_v[bl, t, sl] * NM + m_v[bl, t, sl]

        def start_gather(i, b):
            bl = i // T
            t = i % T
            pltpu.async_copy(hm_hbm.at[c_v.at[bl, t]], rows[b], gsem[b])

        def wait_gather(b):
            pltpu.make_async_copy(
                hm_hbm.at[c_v.at[0, 0]], rows[b], gsem[b]).wait()

        def start_out(i, b):
            bl = i // T
            t = i % T
            pltpu.async_copy(
                rows[b], out_hbm.at[b0 + bl, t, pl.ds(s0, SPW), :], osem[b])

        def wait_out(b):
            pltpu.make_async_copy(
                rows[b], out_hbm.at[0, 0, pl.ds(s0, SPW), :], osem[b]).wait()

        start_gather(0, 0)

        def loop_body(ig, carry):
            for b in range(NBUF):
                i = ig * NBUF + b
                nb = (b + 1) % NBUF
                wait_gather(b)

                @pl.when(i + 1 < NP)
                def _():
                    @pl.when(i >= NBUF - 1)
                    def _():
                        wait_out(nb)
                    start_gather(i + 1, nb)

                @plsc.parallel_loop(0, SPW)
                def add_body(r):
                    for j in range(DVEC):
                        sl = pl.ds(j * _LANES, _LANES)
                        plsc.addupdate(rows[b].at[r, sl], pos_v[r, sl])

                start_out(i, b)
            return carry
        lax.fori_loop(0, NP // NBUF, loop_body, 0)

        for b in range(NBUF):
            wait_out(b)

    return sc_kernel


def kernel(batch_size, total_length, position_ids, week_ids, day_ids,
           hour_ids, minute_ids, device, position_table, week_table,
           day_table, hour_table, minute_table):
    S, D = position_table.shape
    B = hour_ids.shape[0]
    T = hour_ids.shape[1] // S
    NM = minute_table.shape[0]

    hm = _build_hm(hour_table, minute_table)
    h3 = hour_ids.reshape(B, T, S)
    m3 = minute_ids.reshape(B, T, S)

    sc = _make_sc_kernel(B, T, S, D, NM)
    return sc(position_table, h3, m3, hm)


# speedup vs baseline: 7.3758x; 1.6228x over previous
import functools

import jax
import jax.numpy as jnp
from jax import lax
from jax.experimental import pallas as pl
from jax.experimental.pallas import tpu as pltpu
from jax.experimental.pallas import tpu_sc as plsc

_NC = 2
_NS = 16
_LANES = 16


def _hm_body(h_ref, m_ref, o_ref):
    o_ref[...] = h_ref[...][:, None, :] + m_ref[...][None, :, :]


def _build_hm(hour_table, minute_table):
    nh, d = hour_table.shape
    nm = minute_table.shape[0]
    hm = pl.pallas_call(
        _hm_body,
        out_shape=jax.ShapeDtypeStruct((nh, nm, d), jnp.float32),
    )(hour_table, minute_table)
    return hm.reshape(nh * nm, d)


def _make_sc_kernel(B, T, S, D, NH, NM):
    NW = _NC * _NS
    SG = 4
    PG = NW // SG
    SPW = S // SG
    BPW = B // PG
    NP = BPW * T
    NVEC = SPW // _LANES
    DVEC = D // _LANES
    NBUF = 3

    mesh = plsc.VectorSubcoreMesh(
        core_axis_name="c", subcore_axis_name="s",
        num_cores=_NC, num_subcores=_NS)

    @functools.partial(
        pl.kernel,
        out_type=jax.ShapeDtypeStruct((B, T, S, D), jnp.float32),
        mesh=mesh,
        scratch_types=[
            pltpu.VMEM((BPW, T, SPW), jnp.int32),
            pltpu.VMEM((BPW, T, SPW), jnp.int32),
            pltpu.VMEM((BPW, T, SPW), jnp.int32),
            pltpu.VMEM((SPW, D), jnp.float32),
            pltpu.VMEM((NH * NM, D), jnp.float32),
            [pltpu.VMEM((SPW, D), jnp.float32)] * NBUF,
            [pltpu.SemaphoreType.DMA] * NBUF,
        ],
    )
    def sc_kernel(pos_hbm, h_hbm, m_hbm, hm_hbm, out_hbm,
                  h_v, m_v, c_v, pos_v, hm_v, rows, osem):
        cid = lax.axis_index("c")
        sid = lax.axis_index("s")
        wid = sid * _NC + cid
        sg = wid % SG
        pg = wid // SG
        s0 = sg * SPW
        b0 = pg * BPW

        pltpu.sync_copy(hm_hbm, hm_v)
        pltpu.sync_copy(pos_hbm.at[pl.ds(s0, SPW), :], pos_v)
        for bl in range(BPW):
            pltpu.sync_copy(h_hbm.at[b0 + bl, :, pl.ds(s0, SPW)], h_v.at[bl])
            pltpu.sync_copy(m_hbm.at[b0 + bl, :, pl.ds(s0, SPW)], m_v.at[bl])

        @plsc.parallel_loop(0, BPW * T)
        def cid_body(i):
            bl = i // T
            t = i % T
            for j in range(NVEC):
                sl = pl.ds(j * _LANES, _LANES)
                c_v[bl, t, sl] = h_v[bl, t, sl] * NM + m_v[bl, t, sl]

        def start_out(i, b):
            bl = i // T
            t = i % T
            pltpu.async_copy(
                rows[b], out_hbm.at[b0 + bl, t, pl.ds(s0, SPW), :], osem[b])

        def wait_out(b):
            pltpu.make_async_copy(
                rows[b], out_hbm.at[0, 0, pl.ds(s0, SPW), :], osem[b]).wait()

        def loop_body(ig, carry):
            for b in range(NBUF):
                i = ig * NBUF + b
                bl = i // T
                t = i % T

                @pl.when(i >= NBUF)
                def _():
                    wait_out(b)

                @plsc.parallel_loop(0, NVEC)
                def fill_body(g):
                    rv = c_v[bl, t, pl.ds(g * _LANES, _LANES)]
                    for l in range(_LANES):
                        row = rv[l]
                        r = g * _LANES + l
                        for j in range(DVEC):
                            sl = pl.ds(j * _LANES, _LANES)
                            rows[b][r, sl] = hm_v[row, sl] + pos_v[r, sl]

                start_out(i, b)
            return carry
        lax.fori_loop(0, NP // NBUF, loop_body, 0)

        for b in range(NBUF):
            wait_out(b)

    return sc_kernel


def kernel(batch_size, total_length, position_ids, week_ids, day_ids,
           hour_ids, minute_ids, device, position_table, week_table,
           day_table, hour_table, minute_table):
    S, D = position_table.shape
    B = hour_ids.shape[0]
    T = hour_ids.shape[1] // S
    NH = hour_table.shape[0]
    NM = minute_table.shape[0]

    hm = _build_hm(hour_table, minute_table)
    h3 = hour_ids.reshape(B, T, S)
    m3 = minute_ids.reshape(B, T, S)

    sc = _make_sc_kernel(B, T, S, D, NH, NM)
    return sc(position_table, h3, m3, hm)
